# baseline (device time: 13883 ns/iter reference)
import jax
import jax.numpy as jnp
from jax import lax
from jax.experimental import pallas as pl
from jax.experimental.pallas import tpu as pltpu

M = 1024
D = 512
HALF = M // 2
G = HALF // 4


def kernel(partial, gamma):
    x = partial.reshape(M, D)
    g = gamma.reshape(1, D)

    def body(
        x_ref,
        g_ref,
        out_ref,
        mine_ref,
        stage_ref,
        zsend_ref,
        zrecv_ref,
        xrecv_ref,
        yrecv_ref,
        copy_sems,
        zsend_sems,
        zrecv_sems,
        fwd_sems,
        xrecv_sem,
        yrecv_sem,
    ):
        my_x = lax.axis_index("x")
        my_y = lax.axis_index("y")
        my_z = lax.axis_index("z")
        p_z = (my_x, my_y, 1 - my_z)
        n_x = (1 - my_x, my_y, my_z)
        n_y = (my_x, 1 - my_y, my_z)
        q_me = 2 * my_x + my_y
        q_diag = 3 - q_me
        q_xg = 2 * (1 - my_x) + my_y
        q_yg = 2 * my_x + (1 - my_y)
        my_base = my_z * HALF
        peer_base = (1 - my_z) * HALF

        mine_copy = pltpu.make_async_copy(
            x_ref.at[pl.ds(my_base, HALF), :], mine_ref, copy_sems.at[2]
        )
        mine_copy.start()
        stage_copies = []
        for i, q in enumerate((q_me, q_diag)):
            c = pltpu.make_async_copy(
                x_ref.at[pl.ds(peer_base + q * G, G), :],
                stage_ref.at[i],
                copy_sems.at[i],
            )
            c.start()
            stage_copies.append(c)

        barrier_sem = pltpu.get_barrier_semaphore()
        for p in (p_z, n_x, n_y):
            pl.semaphore_signal(
                barrier_sem,
                inc=1,
                device_id=p,
                device_id_type=pl.DeviceIdType.MESH,
            )
        pl.semaphore_wait(barrier_sem, 3)

        zrdmas = []
        for i in range(2):
            stage_copies[i].wait()
            zsend_ref[i] = stage_ref[i].astype(jnp.bfloat16)
            r = pltpu.make_async_remote_copy(
                src_ref=zsend_ref.at[i],
                dst_ref=zrecv_ref.at[i],
                send_sem=zsend_sems.at[i],
                recv_sem=zrecv_sems.at[i],
                device_id=p_z,
                device_id_type=pl.DeviceIdType.MESH,
            )
            r.start()
            zrdmas.append(r)

        zrdmas[0].wait_recv()
        fwd_x = pltpu.make_async_remote_copy(
            src_ref=zrecv_ref.at[0],
            dst_ref=xrecv_ref,
            send_sem=fwd_sems.at[0],
            recv_sem=xrecv_sem,
            device_id=n_x,
            device_id_type=pl.DeviceIdType.MESH,
        )
        fwd_x.start()
        fwd_y = pltpu.make_async_remote_copy(
            src_ref=zrecv_ref.at[0],
            dst_ref=yrecv_ref,
            send_sem=fwd_sems.at[1],
            recv_sem=yrecv_sem,
            device_id=n_y,
            device_id_type=pl.DeviceIdType.MESH,
        )
        fwd_y.start()

        mine_copy.wait()

        def compute_group(q, contrib):
            y = mine_ref[pl.ds(q * G, G), :] + contrib.astype(jnp.float32)
            ms = jnp.mean(y * y, axis=-1, keepdims=True)
            out_ref[pl.ds(q * G, G), :] = y * lax.rsqrt(ms + 1e-6) * g_ref[...]

        compute_group(q_me, zrecv_ref[0])
        zrdmas[1].wait_recv()
        compute_group(q_diag, zrecv_ref[1])
        fwd_x.wait_recv()
        compute_group(q_xg, xrecv_ref[...])
        fwd_y.wait_recv()
        compute_group(q_yg, yrecv_ref[...])

        for r in zrdmas:
            r.wait_send()
        fwd_x.wait_send()
        fwd_y.wait_send()

    return pl.pallas_call(
        body,
        out_shape=jax.ShapeDtypeStruct((HALF, D), jnp.float32),
        in_specs=[
            pl.BlockSpec(memory_space=pl.ANY),
            pl.BlockSpec(memory_space=pltpu.VMEM),
        ],
        out_specs=pl.BlockSpec(memory_space=pltpu.VMEM),
        scratch_shapes=[
            pltpu.VMEM((HALF, D), jnp.float32),
            pltpu.VMEM((2, G, D), jnp.float32),
            pltpu.VMEM((2, G, D), jnp.bfloat16),
            pltpu.VMEM((2, G, D), jnp.bfloat16),
            pltpu.VMEM((G, D), jnp.bfloat16),
            pltpu.VMEM((G, D), jnp.bfloat16),
            pltpu.SemaphoreType.DMA((3,)),
            pltpu.SemaphoreType.DMA((2,)),
            pltpu.SemaphoreType.DMA((2,)),
            pltpu.SemaphoreType.DMA((2,)),
            pltpu.SemaphoreType.DMA,
            pltpu.SemaphoreType.DMA,
        ],
        compiler_params=pltpu.CompilerParams(collective_id=0),
    )(x, g)


# device time: 13249 ns/iter; 1.0479x vs baseline; 1.0479x over previous
import jax
import jax.numpy as jnp
from jax import lax
from jax.experimental import pallas as pl
from jax.experimental.pallas import tpu as pltpu

M = 1024
D = 512
HALF = M // 2
K = 4
CH = HALF // K


def kernel(partial, gamma):
    x = partial.reshape(M, D)
    g = gamma.reshape(1, D)

    def body(
        x_ref,
        g_ref,
        out_hbm,
        mine_ref,
        peer_ref,
        send_ref,
        recv_ref,
        out_vmem,
        copy_sems,
        send_sems,
        recv_sems,
        out_sems,
    ):
        my_x = lax.axis_index("x")
        my_y = lax.axis_index("y")
        my_z = lax.axis_index("z")
        peer = (my_x, my_y, 1 - my_z)
        my_base = my_z * HALF
        peer_base = (1 - my_z) * HALF

        mine_copy = pltpu.make_async_copy(
            x_ref.at[pl.ds(my_base, HALF), :], mine_ref, copy_sems.at[K]
        )
        mine_copy.start()
        peer_copies = []
        for k in range(K):
            c = pltpu.make_async_copy(
                x_ref.at[pl.ds(peer_base + k * CH, CH), :],
                peer_ref.at[k],
                copy_sems.at[k],
            )
            c.start()
            peer_copies.append(c)

        barrier_sem = pltpu.get_barrier_semaphore()
        pl.semaphore_signal(
            barrier_sem,
            inc=1,
            device_id=peer,
            device_id_type=pl.DeviceIdType.MESH,
        )
        pl.semaphore_wait(barrier_sem, 1)

        rdmas = []
        for k in range(K):
            peer_copies[k].wait()
            send_ref[k] = peer_ref[k].astype(jnp.bfloat16)
            r = pltpu.make_async_remote_copy(
                src_ref=send_ref.at[k],
                dst_ref=recv_ref.at[k],
                send_sem=send_sems.at[k],
                recv_sem=recv_sems.at[k],
                device_id=peer,
                device_id_type=pl.DeviceIdType.MESH,
            )
            r.start()
            rdmas.append(r)

        mine_copy.wait()
        out_copies = []
        for k in range(K):
            rdmas[k].wait_recv()
            y = mine_ref[pl.ds(k * CH, CH), :] + recv_ref[k].astype(
                jnp.float32
            )
            ms = jnp.mean(y * y, axis=-1, keepdims=True)
            out_vmem[k] = y * lax.rsqrt(ms + 1e-6) * g_ref[...]
            oc = pltpu.make_async_copy(
                out_vmem.at[k],
                out_hbm.at[pl.ds(k * CH, CH), :],
                out_sems.at[k],
            )
            oc.start()
            out_copies.append(oc)

        for oc in out_copies:
            oc.wait()
        for r in rdmas:
            r.wait_send()

    return pl.pallas_call(
        body,
        out_shape=jax.ShapeDtypeStruct((HALF, D), jnp.float32),
        in_specs=[
            pl.BlockSpec(memory_space=pl.ANY),
            pl.BlockSpec(memory_space=pltpu.VMEM),
        ],
        out_specs=pl.BlockSpec(memory_space=pl.ANY),
        scratch_shapes=[
            pltpu.VMEM((HALF, D), jnp.float32),
            pltpu.VMEM((K, CH, D), jnp.float32),
            pltpu.VMEM((K, CH, D), jnp.bfloat16),
            pltpu.VMEM((K, CH, D), jnp.bfloat16),
            pltpu.VMEM((K, CH, D), jnp.float32),
            pltpu.SemaphoreType.DMA((K + 1,)),
            pltpu.SemaphoreType.DMA((K,)),
            pltpu.SemaphoreType.DMA((K,)),
            pltpu.SemaphoreType.DMA((K,)),
        ],
        compiler_params=pltpu.CompilerParams(collective_id=0),
    )(x, g)
